# baseline (speedup 1.0000x reference)
import functools

import jax
import jax.numpy as jnp
from jax import lax
from jax.experimental import pallas as pl
from jax.experimental.pallas import tpu as pltpu
from jax.experimental.pallas import tpu_sc as plsc

D_MODEL = 64
NUM_CORES = 2
NUM_SUBCORES = 16
NUM_WORKERS = NUM_CORES * NUM_SUBCORES
NBUF = 4


def _gather_rows(weight, token_ids):
    n_b, n_t = token_ids.shape
    rows_per_w = n_b // NUM_WORKERS
    n_groups = rows_per_w // NBUF
    mesh = plsc.VectorSubcoreMesh(core_axis_name="c", subcore_axis_name="s")

    @functools.partial(
        pl.kernel,
        mesh=mesh,
        out_type=jax.ShapeDtypeStruct((n_b, n_t, D_MODEL), jnp.float32),
        scratch_types=[
            pltpu.VMEM((rows_per_w, n_t), jnp.int32),
            *[pltpu.VMEM((n_t, D_MODEL), jnp.float32) for _ in range(NBUF)],
            *[pltpu.SemaphoreType.DMA for _ in range(2 * NBUF)],
        ],
        compiler_params=pltpu.CompilerParams(use_tc_tiling_on_sc=False),
    )
    def k(table_hbm, idx_hbm, out_hbm, idx_v, r0, r1, r2, r3,
          g0, g1, g2, g3, s0, s1, s2, s3):
        rows = (r0, r1, r2, r3)
        gsems = (g0, g1, g2, g3)
        ssems = (s0, s1, s2, s3)
        wid = lax.axis_index("s") * NUM_CORES + lax.axis_index("c")
        base = wid * rows_per_w

        pltpu.sync_copy(idx_hbm.at[pl.ds(base, rows_per_w)], idx_v)

        def gather_start(c, b):
            pltpu.async_copy(table_hbm.at[idx_v.at[c]], rows[b], gsems[b])

        def gather_wait(b):
            pltpu.make_async_copy(table_hbm.at[idx_v.at[0]], rows[b], gsems[b]).wait()

        def store_start(c, b):
            pltpu.async_copy(rows[b], out_hbm.at[base + c], ssems[b])

        def store_wait(b):
            pltpu.make_async_copy(rows[b], out_hbm.at[base], ssems[b]).wait()

        gather_start(0, 0)
        gather_start(1, 1)

        for j in range(NBUF):
            if j >= 2:
                store_wait((j + 2) % NBUF)
            gather_start(j + 2, (j + 2) % NBUF)
            gather_wait(j)
            store_start(j, j)

        def body(g, carry):
            for j in range(NBUF):
                c = g * NBUF + j
                store_wait((j + 2) % NBUF)
                gather_start(c + 2, (j + 2) % NBUF)
                gather_wait(j)
                store_start(c, j)
            return carry

        lax.fori_loop(1, n_groups - 1, body, 0)

        for j in range(NBUF):
            c = (n_groups - 1) * NBUF + j
            if c + 2 < rows_per_w:
                store_wait((j + 2) % NBUF)
                gather_start(c + 2, (j + 2) % NBUF)
            gather_wait(j)
            store_start(c, j)

        for j in range(NBUF):
            store_wait(j)

    return k(weight, token_ids)


def kernel(token_ids, weight):
    return _gather_rows(weight, token_ids.astype(jnp.int32))


# speedup vs baseline: 1.0008x; 1.0008x over previous
import functools

import jax
import jax.numpy as jnp
from jax import lax
from jax.experimental import pallas as pl
from jax.experimental.pallas import tpu as pltpu
from jax.experimental.pallas import tpu_sc as plsc

D_MODEL = 64
NUM_CORES = 2
NUM_SUBCORES = 16
NUM_WORKERS = NUM_CORES * NUM_SUBCORES
NBUF = 8
LA = 4


def _gather_rows(weight, token_ids):
    n_b, n_t = token_ids.shape
    rows_per_w = n_b // NUM_WORKERS
    n_groups = rows_per_w // NBUF
    mesh = plsc.VectorSubcoreMesh(core_axis_name="c", subcore_axis_name="s")

    @functools.partial(
        pl.kernel,
        mesh=mesh,
        out_type=jax.ShapeDtypeStruct((n_b, n_t, D_MODEL), jnp.float32),
        scratch_types=[
            pltpu.VMEM((rows_per_w, n_t), jnp.int32),
            *[pltpu.VMEM((n_t, D_MODEL), jnp.float32) for _ in range(NBUF)],
            *[pltpu.SemaphoreType.DMA for _ in range(2 * NBUF)],
        ],
        compiler_params=pltpu.CompilerParams(use_tc_tiling_on_sc=False),
    )
    def k(table_hbm, idx_hbm, out_hbm, idx_v, *bufs_and_sems):
        rows = bufs_and_sems[:NBUF]
        gsems = bufs_and_sems[NBUF:2 * NBUF]
        ssems = bufs_and_sems[2 * NBUF:]
        wid = lax.axis_index("s") * NUM_CORES + lax.axis_index("c")
        base = wid * rows_per_w

        pltpu.sync_copy(idx_hbm.at[pl.ds(base, rows_per_w)], idx_v)

        def gather_start(c, b):
            pltpu.async_copy(table_hbm.at[idx_v.at[c]], rows[b], gsems[b])

        def gather_wait(b):
            pltpu.make_async_copy(table_hbm.at[idx_v.at[0]], rows[b], gsems[b]).wait()

        def store_start(c, b):
            pltpu.async_copy(rows[b], out_hbm.at[base + c], ssems[b])

        def store_wait(b):
            pltpu.make_async_copy(rows[b], out_hbm.at[base], ssems[b]).wait()

        for j in range(LA):
            gather_start(j, j)

        for j in range(NBUF):
            if j >= NBUF - LA:
                store_wait((j + LA) % NBUF)
            gather_start(j + LA, (j + LA) % NBUF)
            gather_wait(j)
            store_start(j, j)

        def body(g, carry):
            for j in range(NBUF):
                c = g * NBUF + j
                store_wait((j + LA) % NBUF)
                gather_start(c + LA, (j + LA) % NBUF)
                gather_wait(j)
                store_start(c, j)
            return carry

        lax.fori_loop(1, n_groups - 1, body, 0)

        for j in range(NBUF):
            c = (n_groups - 1) * NBUF + j
            if c + LA < rows_per_w:
                store_wait((j + LA) % NBUF)
                gather_start(c + LA, (j + LA) % NBUF)
            gather_wait(j)
            store_start(c, j)

        for j in range(NBUF):
            store_wait(j)

    return k(weight, token_ids)


def kernel(token_ids, weight):
    return _gather_rows(weight, token_ids.astype(jnp.int32))
